# baseline (speedup 1.0000x reference)
import functools

import jax
import jax.numpy as jnp
from jax import lax
from jax.experimental import pallas as pl
from jax.experimental.pallas import tpu as pltpu
from jax.experimental.pallas import tpu_sc as plsc

N_NODES = 10000
D = 128
E = 320000
NC = 2
NS = 16
NW = NC * NS
E_PER_W = E // NW
CHUNK = 80
N_CHUNKS = E_PER_W // CHUNK


def _dot_chunk(rows_u, rows_v, out_all, out_off, tbuf):
  lane = lax.iota(jnp.int32, 16)

  def group_body(g, _):
    gbase = g * 16
    for e in range(16):
      r = gbase + e
      p = rows_u[r, pl.ds(0, 16)] * rows_v[r, pl.ds(0, 16)]
      for j in range(1, D // 16):
        p = p + rows_u[r, pl.ds(j * 16, 16)] * rows_v[r, pl.ds(j * 16, 16)]
      tbuf[e, pl.ds(0, 16)] = p
    out_vec = plsc.load_gather(tbuf, [lane, jnp.zeros((16,), jnp.int32)])
    for c in range(1, 16):
      out_vec = out_vec + plsc.load_gather(
          tbuf, [lane, jnp.full((16,), c, jnp.int32)])
    out_all[pl.ds(out_off + gbase, 16)] = out_vec
    return 0

  lax.fori_loop(0, CHUNK // 16, group_body, 0, unroll=True)


def _slot_types():
  return [
      pltpu.VMEM((CHUNK, D), jnp.float32),
      pltpu.VMEM((CHUNK, D), jnp.float32),
      pltpu.SemaphoreType.DMA,
      pltpu.SemaphoreType.DMA,
  ]


@functools.partial(
    pl.kernel,
    out_type=jax.ShapeDtypeStruct((E,), jnp.float32),
    mesh=plsc.VectorSubcoreMesh(core_axis_name="c", subcore_axis_name="s"),
    compiler_params=pltpu.CompilerParams(needs_layout_passes=False),
    scratch_types=[
        pltpu.VMEM((E_PER_W,), jnp.int32),
        pltpu.VMEM((E_PER_W,), jnp.int32),
        pltpu.VMEM((E_PER_W,), jnp.float32),
        pltpu.VMEM((16, 17), jnp.float32),
    ] + _slot_types() * 2,
)
def _score_kernel(x_hbm, src_hbm, dst_hbm, out_hbm,
                  idx_all_u, idx_all_v, out_all, tbuf, *slot_refs):
  wid = lax.axis_index("s") * NC + lax.axis_index("c")
  base = wid * E_PER_W
  slots = (slot_refs[:4], slot_refs[4:])

  pltpu.sync_copy(src_hbm.at[pl.ds(base, E_PER_W)], idx_all_u)
  pltpu.sync_copy(dst_hbm.at[pl.ds(base, E_PER_W)], idx_all_v)

  def issue(i, s):
    rows_u, rows_v, sem_u, sem_v = s
    pltpu.async_copy(
        x_hbm.at[idx_all_u.at[pl.ds(i * CHUNK, CHUNK)]], rows_u, sem_u)
    pltpu.async_copy(
        x_hbm.at[idx_all_v.at[pl.ds(i * CHUNK, CHUNK)]], rows_v, sem_v)

  def finish(i, s):
    rows_u, rows_v, sem_u, sem_v = s
    pltpu.make_async_copy(
        x_hbm.at[idx_all_u.at[pl.ds(i * CHUNK, CHUNK)]], rows_u, sem_u).wait()
    pltpu.make_async_copy(
        x_hbm.at[idx_all_v.at[pl.ds(i * CHUNK, CHUNK)]], rows_v, sem_v).wait()
    _dot_chunk(rows_u, rows_v, out_all, i * CHUNK, tbuf)

  issue(0, slots[0])

  def pair_body(g, _):
    i = 2 * g
    issue(i + 1, slots[1])
    finish(i, slots[0])
    issue(i + 2, slots[0])
    finish(i + 1, slots[1])
    return 0

  lax.fori_loop(0, (N_CHUNKS - 1) // 2, pair_body, 0)
  finish(N_CHUNKS - 1, slots[0])

  pltpu.sync_copy(out_all, out_hbm.at[pl.ds(base, E_PER_W)])


def kernel(x, edge_index):
  src = edge_index[0].astype(jnp.int32)
  dst = edge_index[1].astype(jnp.int32)
  score = _score_kernel(x, src, dst)
  return score.reshape(E, 1)


# speedup vs baseline: 1.4846x; 1.4846x over previous
import functools

import jax
import jax.numpy as jnp
from jax import lax
from jax.experimental import pallas as pl
from jax.experimental.pallas import tpu as pltpu
from jax.experimental.pallas import tpu_sc as plsc

N_NODES = 10000
D = 128
E = 320000
NC = 2
NS = 16
NW = NC * NS
E_PER_W = E // NW
CHUNK = 80
N_CHUNKS = E_PER_W // CHUNK


def _dot_chunk(rows_u, rows_v, out_all, out_off, tbuf):
  lane = lax.iota(jnp.int32, 16)

  def _tree_sum(vals):
    while len(vals) > 1:
      vals = [a + b for a, b in zip(vals[::2], vals[1::2])]
    return vals[0]

  def group_body(g, _):
    gbase = g * 16
    for e in range(16):
      r = gbase + e
      prods = [
          rows_u[r, pl.ds(j * 16, 16)] * rows_v[r, pl.ds(j * 16, 16)]
          for j in range(D // 16)
      ]
      tbuf[e, pl.ds(0, 16)] = _tree_sum(prods)
    cols = [
        plsc.load_gather(tbuf, [lane, jnp.full((16,), c, jnp.int32)])
        for c in range(16)
    ]
    out_all[pl.ds(out_off + gbase, 16)] = _tree_sum(cols)
    return 0

  lax.fori_loop(0, CHUNK // 16, group_body, 0)


def _slot_types():
  return [
      pltpu.VMEM((CHUNK, D), jnp.float32),
      pltpu.VMEM((CHUNK, D), jnp.float32),
      pltpu.SemaphoreType.DMA,
      pltpu.SemaphoreType.DMA,
  ]


@functools.partial(
    pl.kernel,
    out_type=jax.ShapeDtypeStruct((E,), jnp.float32),
    mesh=plsc.VectorSubcoreMesh(core_axis_name="c", subcore_axis_name="s"),
    compiler_params=pltpu.CompilerParams(needs_layout_passes=False),
    scratch_types=[
        pltpu.VMEM((E_PER_W,), jnp.int32),
        pltpu.VMEM((E_PER_W,), jnp.int32),
        pltpu.VMEM((E_PER_W,), jnp.float32),
        pltpu.VMEM((16, 17), jnp.float32),
    ] + _slot_types() * 2,
)
def _score_kernel(x_hbm, src_hbm, dst_hbm, out_hbm,
                  idx_all_u, idx_all_v, out_all, tbuf, *slot_refs):
  wid = lax.axis_index("s") * NC + lax.axis_index("c")
  base = wid * E_PER_W
  slots = (slot_refs[:4], slot_refs[4:])

  pltpu.sync_copy(src_hbm.at[pl.ds(base, E_PER_W)], idx_all_u)
  pltpu.sync_copy(dst_hbm.at[pl.ds(base, E_PER_W)], idx_all_v)

  def issue(i, s):
    rows_u, rows_v, sem_u, sem_v = s
    pltpu.async_copy(
        x_hbm.at[idx_all_u.at[pl.ds(i * CHUNK, CHUNK)]], rows_u, sem_u)
    pltpu.async_copy(
        x_hbm.at[idx_all_v.at[pl.ds(i * CHUNK, CHUNK)]], rows_v, sem_v)

  def finish(i, s):
    rows_u, rows_v, sem_u, sem_v = s
    pltpu.make_async_copy(
        x_hbm.at[idx_all_u.at[pl.ds(i * CHUNK, CHUNK)]], rows_u, sem_u).wait()
    pltpu.make_async_copy(
        x_hbm.at[idx_all_v.at[pl.ds(i * CHUNK, CHUNK)]], rows_v, sem_v).wait()
    _dot_chunk(rows_u, rows_v, out_all, i * CHUNK, tbuf)

  issue(0, slots[0])

  def pair_body(g, _):
    i = 2 * g
    issue(i + 1, slots[1])
    finish(i, slots[0])
    issue(i + 2, slots[0])
    finish(i + 1, slots[1])
    return 0

  lax.fori_loop(0, (N_CHUNKS - 1) // 2, pair_body, 0)
  finish(N_CHUNKS - 1, slots[0])

  pltpu.sync_copy(out_all, out_hbm.at[pl.ds(base, E_PER_W)])


def kernel(x, edge_index):
  src = edge_index[0].astype(jnp.int32)
  dst = edge_index[1].astype(jnp.int32)
  score = _score_kernel(x, src, dst)
  return score.reshape(E, 1)


# speedup vs baseline: 1.6266x; 1.0956x over previous
import functools

import jax
import jax.numpy as jnp
from jax import lax
from jax.experimental import pallas as pl
from jax.experimental.pallas import tpu as pltpu
from jax.experimental.pallas import tpu_sc as plsc

N_NODES = 10000
D = 128
E = 320000
NC = 2
NS = 16
NW = NC * NS
E_PER_W = E // NW
CHUNK = 80
N_CHUNKS = E_PER_W // CHUNK


def _dot_chunk(rows_u, rows_v, out_all, out_off, tbuf):
  lane = lax.iota(jnp.int32, 16)

  def group_body(g, _):
    gbase = g * 16
    for e in range(16):
      r = gbase + e
      p = rows_u[r, pl.ds(0, 16)] * rows_v[r, pl.ds(0, 16)]
      for j in range(1, D // 16):
        p = p + rows_u[r, pl.ds(j * 16, 16)] * rows_v[r, pl.ds(j * 16, 16)]
      tbuf[e, pl.ds(0, 16)] = p
    out_vec = plsc.load_gather(tbuf, [lane, jnp.zeros((16,), jnp.int32)])
    for c in range(1, 16):
      out_vec = out_vec + plsc.load_gather(
          tbuf, [lane, jnp.full((16,), c, jnp.int32)])
    out_all[pl.ds(out_off + gbase, 16)] = out_vec
    return 0

  lax.fori_loop(0, CHUNK // 16, group_body, 0)


def _slot_types():
  return [
      pltpu.VMEM((CHUNK, D), jnp.float32),
      pltpu.VMEM((CHUNK, D), jnp.float32),
      pltpu.SemaphoreType.DMA,
      pltpu.SemaphoreType.DMA,
  ]


@functools.partial(
    pl.kernel,
    out_type=jax.ShapeDtypeStruct((E,), jnp.float32),
    mesh=plsc.VectorSubcoreMesh(core_axis_name="c", subcore_axis_name="s"),
    compiler_params=pltpu.CompilerParams(needs_layout_passes=False),
    scratch_types=[
        pltpu.VMEM((E_PER_W,), jnp.int32),
        pltpu.VMEM((E_PER_W,), jnp.int32),
        pltpu.VMEM((E_PER_W,), jnp.float32),
        pltpu.VMEM((16, 17), jnp.float32),
    ] + _slot_types() * 2,
)
def _score_kernel(x_hbm, src_hbm, dst_hbm, out_hbm,
                  idx_all_u, idx_all_v, out_all, tbuf, *slot_refs):
  wid = lax.axis_index("s") * NC + lax.axis_index("c")
  base = wid * E_PER_W
  slots = (slot_refs[:4], slot_refs[4:])

  pltpu.sync_copy(src_hbm.at[pl.ds(base, E_PER_W)], idx_all_u)
  pltpu.sync_copy(dst_hbm.at[pl.ds(base, E_PER_W)], idx_all_v)

  def issue(i, s):
    rows_u, rows_v, sem_u, sem_v = s
    pltpu.async_copy(
        x_hbm.at[idx_all_u.at[pl.ds(i * CHUNK, CHUNK)]], rows_u, sem_u)
    pltpu.async_copy(
        x_hbm.at[idx_all_v.at[pl.ds(i * CHUNK, CHUNK)]], rows_v, sem_v)

  def finish(i, s):
    rows_u, rows_v, sem_u, sem_v = s
    pltpu.make_async_copy(
        x_hbm.at[idx_all_u.at[pl.ds(i * CHUNK, CHUNK)]], rows_u, sem_u).wait()
    pltpu.make_async_copy(
        x_hbm.at[idx_all_v.at[pl.ds(i * CHUNK, CHUNK)]], rows_v, sem_v).wait()
    _dot_chunk(rows_u, rows_v, out_all, i * CHUNK, tbuf)

  issue(0, slots[0])

  def pair_body(g, _):
    i = 2 * g
    issue(i + 1, slots[1])
    finish(i, slots[0])
    issue(i + 2, slots[0])
    finish(i + 1, slots[1])
    return 0

  lax.fori_loop(0, (N_CHUNKS - 1) // 2, pair_body, 0)
  finish(N_CHUNKS - 1, slots[0])

  pltpu.sync_copy(out_all, out_hbm.at[pl.ds(base, E_PER_W)])


def kernel(x, edge_index):
  src = edge_index[0].astype(jnp.int32)
  dst = edge_index[1].astype(jnp.int32)
  score = _score_kernel(x, src, dst)
  return score.reshape(E, 1)
